# baseline (speedup 1.0000x reference)
import functools
import math

import jax
import jax.numpy as jnp
from jax.experimental import pallas as pl

B = 16384
BM = 1024
_EPS = 1e-5
_INV_SQRT2 = 1.0 / math.sqrt(2.0)


def _gelu_exact(a):
    return 0.5 * a * (1.0 + jax.lax.erf(a * _INV_SQRT2))


def _mm_stats_body(x_ref, w_ref, h_ref, s_ref, ss_ref):
    h = jnp.dot(x_ref[...], w_ref[...], preferred_element_type=jnp.float32)
    h_ref[...] = h
    ps = jnp.sum(h, axis=0, keepdims=True)
    pss = jnp.sum(h * h, axis=0, keepdims=True)

    @pl.when(pl.program_id(0) == 0)
    def _init():
        s_ref[...] = ps
        ss_ref[...] = pss

    @pl.when(pl.program_id(0) != 0)
    def _acc():
        s_ref[...] += ps
        ss_ref[...] += pss


def _bn_gelu_mm_body(h_ref, s_ref, ss_ref, g_ref, b_ref, w_ref,
                     o_ref, s2_ref, ss2_ref):
    mu = s_ref[...] / B
    var = ss_ref[...] / B - mu * mu
    hn = (h_ref[...] - mu) * jax.lax.rsqrt(var + _EPS)
    a = _gelu_exact(hn * g_ref[...] + b_ref[...])
    h2 = jnp.dot(a, w_ref[...], preferred_element_type=jnp.float32)
    o_ref[...] = h2
    ps = jnp.sum(h2, axis=0, keepdims=True)
    pss = jnp.sum(h2 * h2, axis=0, keepdims=True)

    @pl.when(pl.program_id(0) == 0)
    def _init():
        s2_ref[...] = ps
        ss2_ref[...] = pss

    @pl.when(pl.program_id(0) != 0)
    def _acc():
        s2_ref[...] += ps
        ss2_ref[...] += pss


def _bn_gelu_out_body(h_ref, s_ref, ss_ref, g_ref, b_ref, w_ref, bo_ref, o_ref):
    mu = s_ref[...] / B
    var = ss_ref[...] / B - mu * mu
    hn = (h_ref[...] - mu) * jax.lax.rsqrt(var + _EPS)
    a = _gelu_exact(hn * g_ref[...] + b_ref[...])
    o_ref[...] = jnp.sum(a * w_ref[...], axis=1, keepdims=True) + bo_ref[...]


def _row_spec(d):
    return pl.BlockSpec((BM, d), lambda i: (i, 0))


def _full_spec(shape):
    return pl.BlockSpec(shape, lambda i: (0,) * len(shape))


def _mm_stats(x, w):
    m, k = x.shape
    n = w.shape[1]
    return pl.pallas_call(
        _mm_stats_body,
        grid=(m // BM,),
        in_specs=[_row_spec(k), _full_spec((k, n))],
        out_specs=[_row_spec(n), _full_spec((1, n)), _full_spec((1, n))],
        out_shape=[
            jax.ShapeDtypeStruct((m, n), jnp.float32),
            jax.ShapeDtypeStruct((1, n), jnp.float32),
            jax.ShapeDtypeStruct((1, n), jnp.float32),
        ],
    )(x, w)


def _bn_gelu_mm(h, s, ss, g, b, w):
    m, k = h.shape
    n = w.shape[1]
    return pl.pallas_call(
        _bn_gelu_mm_body,
        grid=(m // BM,),
        in_specs=[_row_spec(k), _full_spec((1, k)), _full_spec((1, k)),
                  _full_spec((1, k)), _full_spec((1, k)), _full_spec((k, n))],
        out_specs=[_row_spec(n), _full_spec((1, n)), _full_spec((1, n))],
        out_shape=[
            jax.ShapeDtypeStruct((m, n), jnp.float32),
            jax.ShapeDtypeStruct((1, n), jnp.float32),
            jax.ShapeDtypeStruct((1, n), jnp.float32),
        ],
    )(h, s, ss, g, b, w)


def _bn_gelu_out(h, s, ss, g, b, w, bo):
    m, k = h.shape
    return pl.pallas_call(
        _bn_gelu_out_body,
        grid=(m // BM,),
        in_specs=[_row_spec(k), _full_spec((1, k)), _full_spec((1, k)),
                  _full_spec((1, k)), _full_spec((1, k)), _full_spec((1, k)),
                  _full_spec((1, 1))],
        out_specs=_row_spec(1),
        out_shape=jax.ShapeDtypeStruct((m, 1), jnp.float32),
    )(h, s, ss, g, b, w, bo)


def kernel(queue_type, region, tier, champion_ids, avg_winrate, avg_kda,
           avg_cs, game_duration_norm, emb_queue_type, emb_region, emb_tier,
           champ_emb, W_num, b_num, W1, g1, beta1, W2, g2, beta2, W3, g3,
           beta3, W_out, b_out):
    bsz = champion_ids.shape[0]
    e1 = jnp.take(emb_queue_type, queue_type, axis=0)
    e2 = jnp.take(emb_region, region, axis=0)
    e3 = jnp.take(emb_tier, tier, axis=0)
    ce = jnp.take(champ_emb, champion_ids, axis=0).reshape(bsz, -1)
    num = jnp.stack([avg_winrate, avg_kda, avg_cs, game_duration_norm], axis=1)
    ne = num @ W_num.T + b_num
    x = jnp.concatenate([e1, e2, e3, ce, ne], axis=1)

    h1, s1, ss1 = _mm_stats(x, W1.T)
    h2, s2, ss2 = _bn_gelu_mm(h1, s1, ss1, g1[None, :], beta1[None, :], W2.T)
    h3, s3, ss3 = _bn_gelu_mm(h2, s2, ss2, g2[None, :], beta2[None, :], W3.T)
    out = _bn_gelu_out(h3, s3, ss3, g3[None, :], beta3[None, :],
                       W_out, b_out[None, :])
    return out[:, 0]


# speedup vs baseline: 2.6303x; 2.6303x over previous
import functools
import math

import jax
import jax.numpy as jnp
from jax import lax
from jax.experimental import pallas as pl
from jax.experimental.pallas import tpu as pltpu
from jax.experimental.pallas import tpu_sc as plsc

B = 16384
BM = 1024
_EPS = 1e-5
_INV_SQRT2 = 1.0 / math.sqrt(2.0)

_NC, _NS = 2, 16
_NW = _NC * _NS
_RPW = B // _NW
_RPC = 8
_IPC = _RPC * 13
_NCH = _RPW // _RPC


_CPS = 16
_NSS = _NCH // _CPS
_HALF = _CPS * _IPC


def _sc_gather_body(idx_hbm, table_hbm, x_hbm, idx_v, bigbuf, gsem):
    wid = lax.axis_index("s") * _NC + lax.axis_index("c")
    pltpu.sync_copy(idx_hbm.at[wid], idx_v)
    out_base = wid * (_RPW * 13)

    def _fire16(t):
        off = (t % 2) * _HALF
        for i in range(_CPS):
            pltpu.async_copy(table_hbm.at[idx_v.at[t * _CPS + i]],
                             bigbuf.at[pl.ds(off + i * _IPC, _IPC), :], gsem)

    def _drain16():
        pltpu.make_async_copy(x_hbm.at[pl.ds(0, _HALF), :],
                              bigbuf.at[pl.ds(0, _HALF), :], gsem).wait()

    def _write(t):
        off = (t % 2) * _HALF
        pltpu.sync_copy(bigbuf.at[pl.ds(off, _HALF), :],
                        x_hbm.at[pl.ds(out_base + t * _HALF, _HALF), :])

    _fire16(0)

    def step(t, carry):
        _fire16(t)
        _drain16()
        _write(t - 1)
        return carry

    lax.fori_loop(1, _NSS, step, 0)
    _drain16()
    _write(_NSS - 1)


@functools.cache
def _sc_gather_fn():
    return pl.kernel(
        _sc_gather_body,
        mesh=plsc.VectorSubcoreMesh(core_axis_name="c", subcore_axis_name="s"),
        compiler_params=pltpu.CompilerParams(use_tc_tiling_on_sc=False),
        out_type=jax.ShapeDtypeStruct((B * 13, 64), jnp.bfloat16),
        scratch_types=[
            pltpu.VMEM((_NCH, _IPC), jnp.int32),
            pltpu.VMEM((2 * _HALF, 64), jnp.bfloat16),
            pltpu.SemaphoreType.DMA,
        ],
    )


def _sc_gather(idx_packed, table):
    return _sc_gather_fn()(idx_packed, table)


def _gelu_exact(a):
    return 0.5 * a * (1.0 + jax.lax.erf(a * _INV_SQRT2))


def _mm1_stats_body(x_ref, num_ref, wn_ref, bn_ref, w_ref, h_ref, s_ref, ss_ref):
    xr = x_ref[...].reshape(BM, 13, 64)
    xc = jnp.concatenate([xr[:, j, :] for j in range(13)], axis=1)
    h = jnp.dot(xc.astype(jnp.float32), w_ref[0:832, :],
                preferred_element_type=jnp.float32)
    ne = lax.dot_general(num_ref[...], wn_ref[...], (((1,), (1,)), ((), ())),
                         preferred_element_type=jnp.float32) + bn_ref[...]
    h = h + jnp.dot(ne, w_ref[832:896, :], preferred_element_type=jnp.float32)
    h_ref[...] = h
    ps = jnp.sum(h, axis=0, keepdims=True)
    pss = jnp.sum(h * h, axis=0, keepdims=True)

    @pl.when(pl.program_id(0) == 0)
    def _init():
        s_ref[...] = ps
        ss_ref[...] = pss

    @pl.when(pl.program_id(0) != 0)
    def _acc():
        s_ref[...] += ps
        ss_ref[...] += pss


def _bn_gelu_mm_body(h_ref, s_ref, ss_ref, g_ref, b_ref, w_ref,
                     o_ref, s2_ref, ss2_ref):
    mu = s_ref[...] / B
    var = ss_ref[...] / B - mu * mu
    hn = (h_ref[...] - mu) * jax.lax.rsqrt(var + _EPS)
    a = _gelu_exact(hn * g_ref[...] + b_ref[...])
    h2 = jnp.dot(a, w_ref[...], preferred_element_type=jnp.float32)
    o_ref[...] = h2
    ps = jnp.sum(h2, axis=0, keepdims=True)
    pss = jnp.sum(h2 * h2, axis=0, keepdims=True)

    @pl.when(pl.program_id(0) == 0)
    def _init():
        s2_ref[...] = ps
        ss2_ref[...] = pss

    @pl.when(pl.program_id(0) != 0)
    def _acc():
        s2_ref[...] += ps
        ss2_ref[...] += pss


def _bn_gelu_out_body(h_ref, s_ref, ss_ref, g_ref, b_ref, w_ref, bo_ref, o_ref):
    mu = s_ref[...] / B
    var = ss_ref[...] / B - mu * mu
    hn = (h_ref[...] - mu) * jax.lax.rsqrt(var + _EPS)
    a = _gelu_exact(hn * g_ref[...] + b_ref[...])
    o_ref[...] = jnp.sum(a * w_ref[...], axis=1, keepdims=True) + bo_ref[...]


def _row_spec(d):
    return pl.BlockSpec((BM, d), lambda i: (i, 0))


def _full_spec(shape):
    return pl.BlockSpec(shape, lambda i: (0,) * len(shape))


def _mm1_stats(x, num, wn, bn, w):
    m = x.shape[0] // 13
    n = w.shape[1]
    return pl.pallas_call(
        _mm1_stats_body,
        grid=(m // BM,),
        in_specs=[pl.BlockSpec((BM * 13, 64), lambda i: (i, 0)),
                  _row_spec(4), _full_spec((64, 4)),
                  _full_spec((1, 64)), _full_spec((896, n))],
        out_specs=[_row_spec(n), _full_spec((1, n)), _full_spec((1, n))],
        out_shape=[
            jax.ShapeDtypeStruct((m, n), jnp.float32),
            jax.ShapeDtypeStruct((1, n), jnp.float32),
            jax.ShapeDtypeStruct((1, n), jnp.float32),
        ],
    )(x, num, wn, bn, w)


def _bn_gelu_mm(h, s, ss, g, b, w):
    m, k = h.shape
    n = w.shape[1]
    return pl.pallas_call(
        _bn_gelu_mm_body,
        grid=(m // BM,),
        in_specs=[_row_spec(k), _full_spec((1, k)), _full_spec((1, k)),
                  _full_spec((1, k)), _full_spec((1, k)), _full_spec((k, n))],
        out_specs=[_row_spec(n), _full_spec((1, n)), _full_spec((1, n))],
        out_shape=[
            jax.ShapeDtypeStruct((m, n), jnp.float32),
            jax.ShapeDtypeStruct((1, n), jnp.float32),
            jax.ShapeDtypeStruct((1, n), jnp.float32),
        ],
    )(h, s, ss, g, b, w)


def _bn_gelu_out(h, s, ss, g, b, w, bo):
    m, k = h.shape
    return pl.pallas_call(
        _bn_gelu_out_body,
        grid=(m // BM,),
        in_specs=[_row_spec(k), _full_spec((1, k)), _full_spec((1, k)),
                  _full_spec((1, k)), _full_spec((1, k)), _full_spec((1, k)),
                  _full_spec((1, 1))],
        out_specs=_row_spec(1),
        out_shape=jax.ShapeDtypeStruct((m, 1), jnp.float32),
    )(h, s, ss, g, b, w, bo)


def kernel(queue_type, region, tier, champion_ids, avg_winrate, avg_kda,
           avg_cs, game_duration_norm, emb_queue_type, emb_region, emb_tier,
           champ_emb, W_num, b_num, W1, g1, beta1, W2, g2, beta2, W3, g3,
           beta3, W_out, b_out):
    idx13 = jnp.stack(
        [queue_type, region + 8, tier + 24]
        + [champion_ids[:, j] + 34 for j in range(10)], axis=1
    ).astype(jnp.int32)
    idx_packed = idx13.reshape(_NW, _NCH, _IPC)

    table = jnp.concatenate(
        [emb_queue_type, emb_region, emb_tier, champ_emb],
        axis=0).astype(jnp.bfloat16)

    x = _sc_gather(idx_packed, table)

    num = jnp.stack([avg_winrate, avg_kda, avg_cs, game_duration_norm], axis=1)

    h1, s1, ss1 = _mm1_stats(x, num, W_num, b_num[None, :], W1.T)
    h2, s2, ss2 = _bn_gelu_mm(h1, s1, ss1, g1[None, :], beta1[None, :], W2.T)
    h3, s3, ss3 = _bn_gelu_mm(h2, s2, ss2, g2[None, :], beta2[None, :], W3.T)
    out = _bn_gelu_out(h3, s3, ss3, g3[None, :], beta3[None, :],
                       W_out, b_out[None, :])
    return out[:, 0]


# speedup vs baseline: 4.5898x; 1.7450x over previous
import functools
import math

import jax
import jax.numpy as jnp
from jax import lax
from jax.experimental import pallas as pl
from jax.experimental.pallas import tpu as pltpu
from jax.experimental.pallas import tpu_sc as plsc

B = 16384
BM = 1024
_EPS = 1e-5
_INV_SQRT2 = 1.0 / math.sqrt(2.0)

_NC, _NS = 2, 16
_NW = _NC * _NS
_RPW = B // _NW
_FPS = 7
_RPC = 16
_IPC = _RPC * _FPS
_NCH = _RPW // _RPC


_CPS = 4
_NSS = _NCH // _CPS
_HALF = _CPS * _IPC


def _sc_gather_body(idx_hbm, table_hbm, x_hbm, idx_v, bigbuf, gsem):
    wid = lax.axis_index("s") * _NC + lax.axis_index("c")
    pltpu.sync_copy(idx_hbm.at[wid], idx_v)
    out_base = wid * (_RPW * _FPS)

    def _fire16(t):
        off = (t % 2) * _HALF
        for i in range(_CPS):
            pltpu.async_copy(table_hbm.at[idx_v.at[t * _CPS + i]],
                             bigbuf.at[pl.ds(off + i * _IPC, _IPC), :], gsem)

    def _drain16():
        pltpu.make_async_copy(x_hbm.at[pl.ds(0, _HALF), :],
                              bigbuf.at[pl.ds(0, _HALF), :], gsem).wait()

    def _write(t):
        off = (t % 2) * _HALF
        pltpu.sync_copy(bigbuf.at[pl.ds(off, _HALF), :],
                        x_hbm.at[pl.ds(out_base + t * _HALF, _HALF), :])

    _fire16(0)

    def step(t, carry):
        _fire16(t)
        _drain16()
        _write(t - 1)
        return carry

    lax.fori_loop(1, _NSS, step, 0)
    _drain16()
    _write(_NSS - 1)


@functools.cache
def _sc_gather_fn():
    return pl.kernel(
        _sc_gather_body,
        mesh=plsc.VectorSubcoreMesh(core_axis_name="c", subcore_axis_name="s"),
        compiler_params=pltpu.CompilerParams(use_tc_tiling_on_sc=True),
        out_type=jax.ShapeDtypeStruct((B * _FPS, 128), jnp.float32),
        scratch_types=[
            pltpu.VMEM((_NCH, _IPC), jnp.int32),
            pltpu.VMEM((2 * _HALF, 128), jnp.float32),
            pltpu.SemaphoreType.DMA,
        ],
    )


def _sc_gather(idx_packed, table):
    return _sc_gather_fn()(idx_packed, table)


def _gelu_exact(a):
    return 0.5 * a * (1.0 + jax.lax.erf(a * _INV_SQRT2))


def _mm1_stats_body(x_ref, num_ref, wn_ref, bn_ref, w_ref, wne_ref,
                    h_ref, s_ref, ss_ref):
    xr = x_ref[...].reshape(BM, _FPS, 128)
    xc = jnp.concatenate([xr[:, j, :] for j in range(_FPS)], axis=1)
    h = jnp.dot(xc, w_ref[...], preferred_element_type=jnp.float32)
    ne = lax.dot_general(num_ref[...], wn_ref[...], (((1,), (1,)), ((), ())),
                         preferred_element_type=jnp.float32) + bn_ref[...]
    h = h + jnp.dot(ne, wne_ref[...], preferred_element_type=jnp.float32)
    h_ref[...] = h
    ps = jnp.sum(h, axis=0, keepdims=True)
    pss = jnp.sum(h * h, axis=0, keepdims=True)

    @pl.when(pl.program_id(0) == 0)
    def _init():
        s_ref[...] = ps
        ss_ref[...] = pss

    @pl.when(pl.program_id(0) != 0)
    def _acc():
        s_ref[...] += ps
        ss_ref[...] += pss


def _bn_gelu_mm_body(h_ref, s_ref, ss_ref, g_ref, b_ref, w_ref,
                     o_ref, s2_ref, ss2_ref):
    mu = s_ref[...] / B
    var = ss_ref[...] / B - mu * mu
    hn = (h_ref[...] - mu) * jax.lax.rsqrt(var + _EPS)
    a = _gelu_exact(hn * g_ref[...] + b_ref[...])
    h2 = jnp.dot(a, w_ref[...], preferred_element_type=jnp.float32)
    o_ref[...] = h2
    ps = jnp.sum(h2, axis=0, keepdims=True)
    pss = jnp.sum(h2 * h2, axis=0, keepdims=True)

    @pl.when(pl.program_id(0) == 0)
    def _init():
        s2_ref[...] = ps
        ss2_ref[...] = pss

    @pl.when(pl.program_id(0) != 0)
    def _acc():
        s2_ref[...] += ps
        ss2_ref[...] += pss


def _bn_gelu_out_body(h_ref, s_ref, ss_ref, g_ref, b_ref, w_ref, bo_ref, o_ref):
    mu = s_ref[...] / B
    var = ss_ref[...] / B - mu * mu
    hn = (h_ref[...] - mu) * jax.lax.rsqrt(var + _EPS)
    a = _gelu_exact(hn * g_ref[...] + b_ref[...])
    o_ref[...] = jnp.sum(a * w_ref[...], axis=1, keepdims=True) + bo_ref[...]


def _row_spec(d):
    return pl.BlockSpec((BM, d), lambda i: (i, 0))


def _full_spec(shape):
    return pl.BlockSpec(shape, lambda i: (0,) * len(shape))


def _mm1_stats(x, num, wn, bn, w, wne):
    m = x.shape[0] // _FPS
    n = w.shape[1]
    return pl.pallas_call(
        _mm1_stats_body,
        grid=(m // BM,),
        in_specs=[pl.BlockSpec((BM * _FPS, 128), lambda i: (i, 0)),
                  _row_spec(4), _full_spec((64, 4)),
                  _full_spec((1, 64)), _full_spec((896, n)),
                  _full_spec((64, n))],
        out_specs=[_row_spec(n), _full_spec((1, n)), _full_spec((1, n))],
        out_shape=[
            jax.ShapeDtypeStruct((m, n), jnp.float32),
            jax.ShapeDtypeStruct((1, n), jnp.float32),
            jax.ShapeDtypeStruct((1, n), jnp.float32),
        ],
    )(x, num, wn, bn, w, wne)


def _bn_gelu_mm(h, s, ss, g, b, w):
    m, k = h.shape
    n = w.shape[1]
    return pl.pallas_call(
        _bn_gelu_mm_body,
        grid=(m // BM,),
        in_specs=[_row_spec(k), _full_spec((1, k)), _full_spec((1, k)),
                  _full_spec((1, k)), _full_spec((1, k)), _full_spec((k, n))],
        out_specs=[_row_spec(n), _full_spec((1, n)), _full_spec((1, n))],
        out_shape=[
            jax.ShapeDtypeStruct((m, n), jnp.float32),
            jax.ShapeDtypeStruct((1, n), jnp.float32),
            jax.ShapeDtypeStruct((1, n), jnp.float32),
        ],
    )(h, s, ss, g, b, w)


def _bn_gelu_out(h, s, ss, g, b, w, bo):
    m, k = h.shape
    return pl.pallas_call(
        _bn_gelu_out_body,
        grid=(m // BM,),
        in_specs=[_row_spec(k), _full_spec((1, k)), _full_spec((1, k)),
                  _full_spec((1, k)), _full_spec((1, k)), _full_spec((1, k)),
                  _full_spec((1, 1))],
        out_specs=_row_spec(1),
        out_shape=jax.ShapeDtypeStruct((m, 1), jnp.float32),
    )(h, s, ss, g, b, w, bo)


def kernel(queue_type, region, tier, champion_ids, avg_winrate, avg_kda,
           avg_cs, game_duration_norm, emb_queue_type, emb_region, emb_tier,
           champ_emb, W_num, b_num, W1, g1, beta1, W2, g2, beta2, W3, g3,
           beta3, W_out, b_out):
    cid = champion_ids.astype(jnp.int32)
    qt, rg, tr = (queue_type.astype(jnp.int32), region.astype(jnp.int32),
                  tier.astype(jnp.int32))
    t_qr = jnp.concatenate(
        [jnp.repeat(emb_queue_type, 16, axis=0), jnp.tile(emb_region, (8, 1))],
        axis=1)
    t_tc = jnp.concatenate(
        [jnp.repeat(emb_tier, 200, axis=0), jnp.tile(champ_emb, (10, 1))],
        axis=1)
    t_cc = jnp.concatenate(
        [jnp.repeat(champ_emb, 200, axis=0), jnp.tile(champ_emb, (200, 1))],
        axis=1)
    t_cq = jnp.concatenate(
        [jnp.repeat(champ_emb, 8, axis=0), jnp.tile(emb_queue_type, (200, 1))],
        axis=1)
    table = jnp.concatenate([t_qr, t_tc, t_cc, t_cq], axis=0)

    idx7 = jnp.stack([
        qt * 16 + rg,
        128 + tr * 200 + cid[:, 0],
        2128 + cid[:, 1] * 200 + cid[:, 2],
        2128 + cid[:, 3] * 200 + cid[:, 4],
        2128 + cid[:, 5] * 200 + cid[:, 6],
        2128 + cid[:, 7] * 200 + cid[:, 8],
        42128 + cid[:, 9] * 8 + qt,
    ], axis=1)
    idx_packed = idx7.reshape(_NW, _NCH, _IPC)

    x = _sc_gather(idx_packed, table)

    num = jnp.stack([avg_winrate, avg_kda, avg_cs, game_duration_norm], axis=1)

    w1t = W1.T
    w1z = jnp.concatenate([w1t[:832], jnp.zeros((64, 256), jnp.float32)])
    h1, s1, ss1 = _mm1_stats(x, num, W_num, b_num[None, :], w1z, w1t[832:])
    h2, s2, ss2 = _bn_gelu_mm(h1, s1, ss1, g1[None, :], beta1[None, :], W2.T)
    h3, s3, ss3 = _bn_gelu_mm(h2, s2, ss2, g2[None, :], beta2[None, :], W3.T)
    out = _bn_gelu_out(h3, s3, ss3, g3[None, :], beta3[None, :],
                       W_out, b_out[None, :])
    return out[:, 0]


# speedup vs baseline: 4.7857x; 1.0427x over previous
import functools
import math

import jax
import jax.numpy as jnp
from jax import lax
from jax.experimental import pallas as pl
from jax.experimental.pallas import tpu as pltpu
from jax.experimental.pallas import tpu_sc as plsc

B = 16384
BM = 2048
_EPS = 1e-5
_INV_SQRT2 = 1.0 / math.sqrt(2.0)

_NC, _NS = 2, 16
_NW = _NC * _NS
_RPW = B // _NW
_FPS = 7
_RPC = 16
_IPC = _RPC * _FPS
_NCH = _RPW // _RPC


_CPS = 4
_NSS = _NCH // _CPS
_HALF = _CPS * _IPC


def _sc_gather_body(idx_hbm, table_hbm, x_hbm, idx_v, bigbuf, gsem):
    wid = lax.axis_index("s") * _NC + lax.axis_index("c")
    pltpu.sync_copy(idx_hbm.at[wid], idx_v)
    out_base = wid * (_RPW * _FPS)

    def _fire16(t):
        off = (t % 2) * _HALF
        for i in range(_CPS):
            pltpu.async_copy(table_hbm.at[idx_v.at[t * _CPS + i]],
                             bigbuf.at[pl.ds(off + i * _IPC, _IPC), :], gsem)

    def _drain16():
        pltpu.make_async_copy(x_hbm.at[pl.ds(0, _HALF), :],
                              bigbuf.at[pl.ds(0, _HALF), :], gsem).wait()

    def _write(t):
        off = (t % 2) * _HALF
        pltpu.sync_copy(bigbuf.at[pl.ds(off, _HALF), :],
                        x_hbm.at[pl.ds(out_base + t * _HALF, _HALF), :])

    _fire16(0)

    def step(t, carry):
        _fire16(t)
        _drain16()
        _write(t - 1)
        return carry

    lax.fori_loop(1, _NSS, step, 0)
    _drain16()
    _write(_NSS - 1)


@functools.cache
def _sc_gather_fn():
    return pl.kernel(
        _sc_gather_body,
        mesh=plsc.VectorSubcoreMesh(core_axis_name="c", subcore_axis_name="s"),
        compiler_params=pltpu.CompilerParams(use_tc_tiling_on_sc=True),
        out_type=jax.ShapeDtypeStruct((B * _FPS, 128), jnp.float32),
        scratch_types=[
            pltpu.VMEM((_NCH, _IPC), jnp.int32),
            pltpu.VMEM((2 * _HALF, 128), jnp.float32),
            pltpu.SemaphoreType.DMA,
        ],
    )


def _sc_gather(idx_packed, table):
    return _sc_gather_fn()(idx_packed, table)


def _gelu_exact(a):
    return 0.5 * a * (1.0 + jax.lax.erf(a * _INV_SQRT2))


def _mm1_stats_body(x_ref, num_ref, wn_ref, bn_ref, w_ref, wne_ref,
                    h_ref, s_ref, ss_ref):
    xr = x_ref[...].reshape(BM, _FPS, 128)
    xc = jnp.concatenate([xr[:, j, :] for j in range(_FPS)], axis=1)
    h = jnp.dot(xc.astype(jnp.bfloat16), w_ref[...],
                preferred_element_type=jnp.float32)
    ne = lax.dot_general(num_ref[...], wn_ref[...], (((1,), (1,)), ((), ())),
                         preferred_element_type=jnp.float32) + bn_ref[...]
    h = h + jnp.dot(ne, wne_ref[...], preferred_element_type=jnp.float32)
    h_ref[...] = h
    ps = jnp.sum(h, axis=0, keepdims=True)
    pss = jnp.sum(h * h, axis=0, keepdims=True)

    @pl.when(pl.program_id(0) == 0)
    def _init():
        s_ref[...] = ps
        ss_ref[...] = pss

    @pl.when(pl.program_id(0) != 0)
    def _acc():
        s_ref[...] += ps
        ss_ref[...] += pss


def _bn_gelu_mm_body(h_ref, s_ref, ss_ref, g_ref, b_ref, w_ref,
                     o_ref, s2_ref, ss2_ref):
    mu = s_ref[...] / B
    var = ss_ref[...] / B - mu * mu
    hn = (h_ref[...] - mu) * jax.lax.rsqrt(var + _EPS)
    a = _gelu_exact(hn * g_ref[...] + b_ref[...])
    h2 = jnp.dot(a, w_ref[...], preferred_element_type=jnp.float32)
    o_ref[...] = h2
    ps = jnp.sum(h2, axis=0, keepdims=True)
    pss = jnp.sum(h2 * h2, axis=0, keepdims=True)

    @pl.when(pl.program_id(0) == 0)
    def _init():
        s2_ref[...] = ps
        ss2_ref[...] = pss

    @pl.when(pl.program_id(0) != 0)
    def _acc():
        s2_ref[...] += ps
        ss2_ref[...] += pss


def _bn_gelu_out_body(h_ref, s_ref, ss_ref, g_ref, b_ref, w_ref, bo_ref, o_ref):
    mu = s_ref[...] / B
    var = ss_ref[...] / B - mu * mu
    hn = (h_ref[...] - mu) * jax.lax.rsqrt(var + _EPS)
    a = _gelu_exact(hn * g_ref[...] + b_ref[...])
    o_ref[...] = jnp.sum(a * w_ref[...], axis=1, keepdims=True) + bo_ref[...]


def _row_spec(d):
    return pl.BlockSpec((BM, d), lambda i: (i, 0))


def _full_spec(shape):
    return pl.BlockSpec(shape, lambda i: (0,) * len(shape))


def _mm1_stats(x, num, wn, bn, w, wne):
    m = x.shape[0] // _FPS
    n = w.shape[1]
    return pl.pallas_call(
        _mm1_stats_body,
        grid=(m // BM,),
        in_specs=[pl.BlockSpec((BM * _FPS, 128), lambda i: (i, 0)),
                  _row_spec(4), _full_spec((64, 4)),
                  _full_spec((1, 64)), _full_spec((896, n)),
                  _full_spec((64, n))],
        out_specs=[_row_spec(n), _full_spec((1, n)), _full_spec((1, n))],
        out_shape=[
            jax.ShapeDtypeStruct((m, n), jnp.float32),
            jax.ShapeDtypeStruct((1, n), jnp.float32),
            jax.ShapeDtypeStruct((1, n), jnp.float32),
        ],
    )(x, num, wn, bn, w, wne)


def _bn_gelu_mm(h, s, ss, g, b, w):
    m, k = h.shape
    n = w.shape[1]
    return pl.pallas_call(
        _bn_gelu_mm_body,
        grid=(m // BM,),
        in_specs=[_row_spec(k), _full_spec((1, k)), _full_spec((1, k)),
                  _full_spec((1, k)), _full_spec((1, k)), _full_spec((k, n))],
        out_specs=[_row_spec(n), _full_spec((1, n)), _full_spec((1, n))],
        out_shape=[
            jax.ShapeDtypeStruct((m, n), jnp.float32),
            jax.ShapeDtypeStruct((1, n), jnp.float32),
            jax.ShapeDtypeStruct((1, n), jnp.float32),
        ],
    )(h, s, ss, g, b, w)


def _bn_gelu_out(h, s, ss, g, b, w, bo):
    m, k = h.shape
    return pl.pallas_call(
        _bn_gelu_out_body,
        grid=(m // BM,),
        in_specs=[_row_spec(k), _full_spec((1, k)), _full_spec((1, k)),
                  _full_spec((1, k)), _full_spec((1, k)), _full_spec((1, k)),
                  _full_spec((1, 1))],
        out_specs=_row_spec(1),
        out_shape=jax.ShapeDtypeStruct((m, 1), jnp.float32),
    )(h, s, ss, g, b, w, bo)


def kernel(queue_type, region, tier, champion_ids, avg_winrate, avg_kda,
           avg_cs, game_duration_norm, emb_queue_type, emb_region, emb_tier,
           champ_emb, W_num, b_num, W1, g1, beta1, W2, g2, beta2, W3, g3,
           beta3, W_out, b_out):
    cid = champion_ids.astype(jnp.int32)
    qt, rg, tr = (queue_type.astype(jnp.int32), region.astype(jnp.int32),
                  tier.astype(jnp.int32))
    t_qr = jnp.concatenate(
        [jnp.repeat(emb_queue_type, 16, axis=0), jnp.tile(emb_region, (8, 1))],
        axis=1)
    t_tc = jnp.concatenate(
        [jnp.repeat(emb_tier, 200, axis=0), jnp.tile(champ_emb, (10, 1))],
        axis=1)
    t_cc = jnp.concatenate(
        [jnp.repeat(champ_emb, 200, axis=0), jnp.tile(champ_emb, (200, 1))],
        axis=1)
    t_cq = jnp.concatenate(
        [jnp.repeat(champ_emb, 8, axis=0), jnp.tile(emb_queue_type, (200, 1))],
        axis=1)
    table = jnp.concatenate([t_qr, t_tc, t_cc, t_cq], axis=0)

    idx7 = jnp.stack([
        qt * 16 + rg,
        128 + tr * 200 + cid[:, 0],
        2128 + cid[:, 1] * 200 + cid[:, 2],
        2128 + cid[:, 3] * 200 + cid[:, 4],
        2128 + cid[:, 5] * 200 + cid[:, 6],
        2128 + cid[:, 7] * 200 + cid[:, 8],
        42128 + cid[:, 9] * 8 + qt,
    ], axis=1)
    idx_packed = idx7.reshape(_NW, _NCH, _IPC)

    x = _sc_gather(idx_packed, table)

    num = jnp.stack([avg_winrate, avg_kda, avg_cs, game_duration_norm], axis=1)

    w1t = W1.T
    w1z = jnp.concatenate([w1t[:832], jnp.zeros((64, 256), jnp.float32)]
                          ).astype(jnp.bfloat16)
    h1, s1, ss1 = _mm1_stats(x, num, W_num, b_num[None, :], w1z, w1t[832:])
    h2, s2, ss2 = _bn_gelu_mm(h1, s1, ss1, g1[None, :], beta1[None, :], W2.T)
    h3, s3, ss3 = _bn_gelu_mm(h2, s2, ss2, g2[None, :], beta2[None, :], W3.T)
    out = _bn_gelu_out(h3, s3, ss3, g3[None, :], beta3[None, :],
                       W_out, b_out[None, :])
    return out[:, 0]
